# Initial kernel scaffold; baseline (speedup 1.0000x reference)
import jax
import jax.numpy as jnp
from jax.experimental import pallas as pl


def kernel(query, xyz, pcl_feat, W_pos, b_pos, W0, b0, W1, b1, W2, b2, W3, b3, W4, b4, Wout, bout):
    raise NotImplementedError("write your pallas kernel here")


# speedup vs baseline: 22.1971x; 22.1971x over previous
import functools

import jax
import jax.numpy as jnp
from jax import lax
from jax.experimental import pallas as pl
from jax.experimental.pallas import tpu as pltpu
from jax.experimental.pallas import tpu_sc as plsc

_NC = 2
_NS = 16
_NW = _NC * _NS


def _bf(v):
    return v.astype(jnp.bfloat16).astype(jnp.float32)


def _knn_body(k_nn, n, q_ref, xt_ref, idx_ref):
    b = pl.program_id(0)
    q = q_ref[0]
    qx, qy, qz = q[:, 0:1], q[:, 1:2], q[:, 2:3]
    xt = xt_ref[0]
    x0, x1, x2 = xt[0:1, :], xt[1:2, :], xt[2:3, :]

    qn = qx * qx + qy * qy + qz * qz
    xn = x0 * x0 + x1 * x1 + x2 * x2
    dot = (_bf(qx) * _bf(x0) + _bf(qy) * _bf(x1) + _bf(qz) * _bf(x2))
    d2 = (qn - 2.0 * dot) + xn

    qb = q.shape[0]
    iota = lax.broadcasted_iota(jnp.int32, (qb, n), 1)
    cols = []
    for _ in range(k_nn):
        m = jnp.min(d2, axis=1, keepdims=True)
        ik = jnp.min(jnp.where(d2 == m, iota, n), axis=1, keepdims=True)
        cols.append(ik)
        d2 = jnp.where(iota == ik, jnp.inf, d2)
    idx_ref[0] = jnp.concatenate(cols, axis=1) + b * n


def _knn(query, xyzt, k_nn, qb):
    b_dim, q_dim, _ = query.shape
    n = xyzt.shape[2]
    return pl.pallas_call(
        functools.partial(_knn_body, k_nn, n),
        grid=(b_dim, q_dim // qb),
        in_specs=[
            pl.BlockSpec((1, qb, 3), lambda b, q: (b, q, 0)),
            pl.BlockSpec((1, 3, n), lambda b, q: (b, 0, 0)),
        ],
        out_specs=pl.BlockSpec((1, qb, k_nn), lambda b, q: (b, q, 0)),
        out_shape=jax.ShapeDtypeStruct((b_dim, q_dim, k_nn), jnp.int32),
    )(query, xyzt)


def _sc_gather(idx_flat, tp, tf):
    total = idx_flat.shape[0]
    per_w = total // _NW
    chunk = 256
    nchunks = per_w // chunk
    dp = tp.shape[1]
    df = tf.shape[1]
    mesh = plsc.VectorSubcoreMesh(core_axis_name="c", subcore_axis_name="s")

    @functools.partial(
        pl.kernel,
        out_type=[
            jax.ShapeDtypeStruct((total, dp), jnp.float32),
            jax.ShapeDtypeStruct((total, df), jnp.float32),
        ],
        mesh=mesh,
        scratch_types=[
            pltpu.VMEM((per_w,), jnp.int32),
            pltpu.VMEM((chunk, dp), jnp.float32),
            pltpu.VMEM((chunk, df), jnp.float32),
            pltpu.SemaphoreType.DMA,
            pltpu.SemaphoreType.DMA,
        ],
    )
    def gather_k(idx_hbm, tp_hbm, tf_hbm, gp_hbm, gf_hbm,
                 idx_v, bufp, buff, semp, semf):
        wid = lax.axis_index("s") * _NC + lax.axis_index("c")
        base = wid * per_w
        pltpu.sync_copy(idx_hbm.at[pl.ds(base, per_w)], idx_v)
        for c in range(nchunks):
            sl = idx_v.at[pl.ds(c * chunk, chunk)]
            cp_p = pltpu.async_copy(tp_hbm.at[sl], bufp, semp)
            cp_f = pltpu.async_copy(tf_hbm.at[sl], buff, semf)
            cp_p.wait()
            cp_f.wait()
            pltpu.sync_copy(bufp, gp_hbm.at[pl.ds(base + c * chunk, chunk)])
            pltpu.sync_copy(buff, gf_hbm.at[pl.ds(base + c * chunk, chunk)])

    return gather_k(idx_flat, tp, tf)


def _mlp_body(k_nn, dp, pos, q_ref, gp_ref, gf_ref, wpos_ref, bpos_ref,
              w0_ref, b0_ref, w1_ref, b1_ref, w2_ref, b2_ref,
              w3_ref, b3_ref, w4_ref, b4_ref, wout_ref, out_ref):
    q = q_ref[...]
    qx, qy, qz = q[:, 0:1], q[:, 1:2], q[:, 2:3]
    gp = gp_ref[...]
    gf = gf_ref[...]
    wpos = wpos_ref[...].astype(jnp.float32)
    bpos = bpos_ref[...]

    pieces = []
    for k in range(k_nn):
        px = gp[:, k * dp:k * dp + 1]
        py = gp[:, k * dp + 1:k * dp + 2]
        pz = gp[:, k * dp + 2:k * dp + 3]
        ins = (qx, qy, qz, px, py, pz, px - qx, py - qy, pz - qz)
        acc = bpos
        for d in range(9):
            acc = acc + _bf(ins[d]) * wpos[d:d + 1, :]
        posk = jnp.maximum(acc, 0.0).astype(jnp.bfloat16)
        pieces.append(posk)
        pieces.append(gf[:, k * 128:(k + 1) * 128].astype(jnp.bfloat16))
    x = jnp.concatenate(pieces, axis=1)

    def layer(h, w_ref, b_ref):
        hh = jnp.dot(h.astype(jnp.bfloat16), w_ref[...],
                     preferred_element_type=jnp.float32)
        return jnp.maximum(hh + b_ref[...], 0.0)

    h = layer(x, w0_ref, b0_ref)
    h = layer(h, w1_ref, b1_ref)
    h = layer(h, w2_ref, b2_ref)
    h = layer(h, w3_ref, b3_ref)
    h = layer(h, w4_ref, b4_ref)
    wo = wout_ref[...].astype(jnp.float32)
    out_ref[...] = jnp.sum(_bf(h) * wo, axis=1, keepdims=True)


def _mlp(query2d, gp, gf, wpos, bpos, w0, b0, w1, b1, w2, b2, w3, b3,
         w4, b4, wout_row, k_nn, dp, qb2):
    bq = query2d.shape[0]
    pos = wpos.shape[1]
    cp = gp.shape[1]
    cf = gf.shape[1]
    c0 = w0.shape[0]
    e2 = w0.shape[1]
    e = w1.shape[1]
    full = lambda shape: pl.BlockSpec(shape, lambda i: tuple(0 for _ in shape))
    return pl.pallas_call(
        functools.partial(_mlp_body, k_nn, dp, pos),
        grid=(bq // qb2,),
        in_specs=[
            pl.BlockSpec((qb2, 3), lambda i: (i, 0)),
            pl.BlockSpec((qb2, cp), lambda i: (i, 0)),
            pl.BlockSpec((qb2, cf), lambda i: (i, 0)),
            full((9, pos)), full((1, pos)),
            full((c0, e2)), full((1, e2)),
            full((e2, e)), full((1, e)),
            full((e, e)), full((1, e)),
            full((e, e)), full((1, e)),
            full((e, e)), full((1, e)),
            full((1, e)),
        ],
        out_specs=pl.BlockSpec((qb2, 1), lambda i: (i, 0)),
        out_shape=jax.ShapeDtypeStruct((bq, 1), jnp.float32),
    )(query2d, gp, gf, wpos, bpos, w0, b0, w1, b1, w2, b2, w3, b3, w4,
      b4, wout_row)


def kernel(query, xyz, pcl_feat, W_pos, b_pos, W0, b0, W1, b1, W2, b2,
           W3, b3, W4, b4, Wout, bout):
    b_dim, q_dim, _ = query.shape
    n = xyz.shape[1]
    out_dim = pcl_feat.shape[-1]
    pos = W_pos.shape[1]
    k_nn = W0.shape[0] // (pos + out_dim)
    dp = 128

    xyzt = jnp.transpose(xyz, (0, 2, 1))
    idx = _knn(query, xyzt, k_nn, qb=128)

    xyzp = jnp.pad(xyz.reshape(b_dim * n, 3), ((0, 0), (0, dp - 3)))
    gp, gf = _sc_gather(
        idx.reshape(-1),
        xyzp,
        pcl_feat.reshape(b_dim * n, out_dim),
    )

    bq = b_dim * q_dim
    out = _mlp(
        query.reshape(bq, 3),
        gp.reshape(bq, k_nn * dp),
        gf.reshape(bq, k_nn * out_dim),
        W_pos.astype(jnp.bfloat16), b_pos.reshape(1, pos),
        W0.astype(jnp.bfloat16), b0.reshape(1, -1),
        W1.astype(jnp.bfloat16), b1.reshape(1, -1),
        W2.astype(jnp.bfloat16), b2.reshape(1, -1),
        W3.astype(jnp.bfloat16), b3.reshape(1, -1),
        W4.astype(jnp.bfloat16), b4.reshape(1, -1),
        Wout.reshape(1, -1).astype(jnp.bfloat16),
        k_nn, dp, qb2=256,
    )
    return (out + bout).reshape(b_dim, q_dim, 1)


# speedup vs baseline: 39.8117x; 1.7936x over previous
import functools

import jax
import jax.numpy as jnp
from jax import lax
from jax.experimental import pallas as pl
from jax.experimental.pallas import tpu as pltpu
from jax.experimental.pallas import tpu_sc as plsc

_NC = 2
_NS = 16
_NW = _NC * _NS


def _bf(v):
    return v.astype(jnp.bfloat16).astype(jnp.float32)


def _knn_body(k_nn, n, base, q_ref, xt_ref, idx_ref):
    q = q_ref[0]
    qx, qy, qz = q[:, 0:1], q[:, 1:2], q[:, 2:3]
    xt = xt_ref[0]
    x0, x1, x2 = xt[0:1, :], xt[1:2, :], xt[2:3, :]

    qn = qx * qx + qy * qy + qz * qz
    xn = x0 * x0 + x1 * x1 + x2 * x2
    dot = jnp.dot(q.astype(jnp.bfloat16), xt.astype(jnp.bfloat16),
                  preferred_element_type=jnp.float32)
    d2 = (qn - 2.0 * dot) + xn

    qb = q.shape[0]
    inf = jnp.float32(jnp.inf)

    f_ch = 16
    s_sl = n // f_ch
    parts = [d2[:, f * s_sl:(f + 1) * s_sl] for f in range(f_ch)]

    def running_min(masked_parts):
        v = masked_parts[0]
        c = jnp.zeros((qb, s_sl), jnp.int32)
        for f in range(1, f_ch):
            lt = masked_parts[f] < v
            v = jnp.where(lt, masked_parts[f], v)
            c = jnp.where(lt, jnp.int32(f), c)
        return v, c

    v1, c1 = running_min(parts)
    parts2 = [jnp.where(c1 == f, inf, parts[f]) for f in range(f_ch)]
    v2, c2 = running_min(parts2)
    parts3 = [jnp.where(c2 == f, inf, parts2[f]) for f in range(f_ch)]
    v3, c3 = running_min(parts3)

    si = lax.broadcasted_iota(jnp.int32, (1, s_sl), 1)
    a_v, b_v, c_v = v1, v2, v3
    a_g = c1 * s_sl + si
    b_g = c2 * s_sl + si
    c_g = c3 * s_sl + si

    sent = jnp.int32(2 * n)
    depth = jnp.zeros((qb, s_sl), jnp.int32)
    bad = jnp.zeros((qb, 1), jnp.bool_)
    cols = []
    for j in range(k_nn):
        m = jnp.min(a_v, axis=1, keepdims=True)
        g = jnp.min(jnp.where(a_v == m, a_g, sent), axis=1, keepdims=True)
        cols.append(g)
        sel = (a_g == g) & (a_v == m)
        if j < k_nn - 1:
            bad = bad | jnp.any(sel & (depth == 2), axis=1, keepdims=True)
            depth = depth + sel.astype(jnp.int32)
            a_v = jnp.where(sel, b_v, a_v)
            a_g = jnp.where(sel, b_g, a_g)
            b_v = jnp.where(sel, c_v, b_v)
            b_g = jnp.where(sel, c_g, b_g)
            c_v = jnp.where(sel, inf, c_v)
    fast = jnp.concatenate(cols, axis=1)

    def slow_path(_):
        dd = d2
        iota = lax.broadcasted_iota(jnp.int32, (qb, n), 1)
        cc = []
        for _ in range(k_nn):
            ik = jnp.argmin(dd, axis=1).reshape(qb, 1)
            cc.append(ik)
            dd = jnp.where(iota == ik, inf, dd)
        return jnp.concatenate(cc, axis=1)

    idxs = lax.cond(jnp.any(bad), slow_path, lambda _: fast, None)
    idx_ref[0] = jnp.transpose(idxs + base, (1, 0))


def _knn(query, xyzt, k_nn, qb, base):
    b_dim, q_dim, _ = query.shape
    n = xyzt.shape[2]
    return pl.pallas_call(
        functools.partial(_knn_body, k_nn, n, base),
        grid=(b_dim, q_dim // qb),
        in_specs=[
            pl.BlockSpec((1, qb, 3), lambda b, q: (b, q, 0)),
            pl.BlockSpec((1, 3, n), lambda b, q: (b, 0, 0)),
        ],
        out_specs=pl.BlockSpec((1, k_nn, qb), lambda b, q: (b, 0, q)),
        out_shape=jax.ShapeDtypeStruct((b_dim, k_nn, q_dim), jnp.int32),
    )(query, xyzt)


def _sc_gather(idx_flat, tp, tf):
    total = idx_flat.shape[0]
    per_w = total // _NW
    chunk = 256
    nchunks = per_w // chunk
    dp = tp.shape[1]
    df = tf.shape[1]
    mesh = plsc.VectorSubcoreMesh(core_axis_name="c", subcore_axis_name="s")

    @functools.partial(
        pl.kernel,
        out_type=[
            jax.ShapeDtypeStruct((total, dp), jnp.float32),
            jax.ShapeDtypeStruct((total, df), jnp.float32),
        ],
        mesh=mesh,
        scratch_types=[
            pltpu.VMEM((per_w,), jnp.int32),
            pltpu.VMEM((chunk, dp), jnp.float32),
            pltpu.VMEM((chunk, df), jnp.float32),
            pltpu.SemaphoreType.DMA,
            pltpu.SemaphoreType.DMA,
        ],
    )
    def gather_k(idx_hbm, tp_hbm, tf_hbm, gp_hbm, gf_hbm,
                 idx_v, bufp, buff, semp, semf):
        wid = lax.axis_index("s") * _NC + lax.axis_index("c")
        base = wid * per_w
        pltpu.sync_copy(idx_hbm.at[pl.ds(base, per_w)], idx_v)
        for c in range(nchunks):
            sl = idx_v.at[pl.ds(c * chunk, chunk)]
            cp_p = pltpu.async_copy(tp_hbm.at[sl], bufp, semp)
            cp_f = pltpu.async_copy(tf_hbm.at[sl], buff, semf)
            cp_p.wait()
            cp_f.wait()
            pltpu.sync_copy(bufp, gp_hbm.at[pl.ds(base + c * chunk, chunk)])
            pltpu.sync_copy(buff, gf_hbm.at[pl.ds(base + c * chunk, chunk)])

    return gather_k(idx_flat, tp, tf)


def _mlp_body(k_nn, dp, pos, q_ref, gp_ref, gf_ref, wpos_ref, bpos_ref,
              w0_ref, b0_ref, w1_ref, b1_ref, w2_ref, b2_ref,
              w3_ref, b3_ref, w4_ref, b4_ref, wout_ref, out_ref):
    q = q_ref[0]
    qx, qy, qz = q[:, 0:1], q[:, 1:2], q[:, 2:3]
    wpos = wpos_ref[...].astype(jnp.float32)
    bpos = bpos_ref[...]

    pieces = []
    for k in range(k_nn):
        gpk = gp_ref[0, k]
        px, py, pz = gpk[:, 0:1], gpk[:, 1:2], gpk[:, 2:3]
        ins = (qx, qy, qz, px, py, pz, px - qx, py - qy, pz - qz)
        acc = bpos
        for d in range(9):
            acc = acc + _bf(ins[d]) * wpos[d:d + 1, :]
        posk = jnp.maximum(acc, 0.0).astype(jnp.bfloat16)
        pieces.append(posk)
        pieces.append(gf_ref[0, k].astype(jnp.bfloat16))
    x = jnp.concatenate(pieces, axis=1)

    def layer(h, w_ref, b_ref):
        hh = jnp.dot(h.astype(jnp.bfloat16), w_ref[...],
                     preferred_element_type=jnp.float32)
        return jnp.maximum(hh + b_ref[...], 0.0)

    h = layer(x, w0_ref, b0_ref)
    h = layer(h, w1_ref, b1_ref)
    h = layer(h, w2_ref, b2_ref)
    h = layer(h, w3_ref, b3_ref)
    h = layer(h, w4_ref, b4_ref)
    wo = wout_ref[...].astype(jnp.float32)
    out_ref[0] = jnp.sum(_bf(h) * wo, axis=1, keepdims=True)


def _mlp(query, gp4, gf4, wpos, bpos, w0, b0, w1, b1, w2, b2, w3, b3,
         w4, b4, wout_row, k_nn, dp, qb2):
    b_dim, q_dim, _ = query.shape
    pos = wpos.shape[1]
    df = gf4.shape[3]
    c0 = w0.shape[0]
    e2 = w0.shape[1]
    e = w1.shape[1]
    nq = q_dim // qb2
    full = lambda shape: pl.BlockSpec(shape, lambda i: tuple(0 for _ in shape))
    return pl.pallas_call(
        functools.partial(_mlp_body, k_nn, dp, pos),
        grid=(b_dim * nq,),
        in_specs=[
            pl.BlockSpec((1, qb2, 3), lambda i: (i // nq, i % nq, 0)),
            pl.BlockSpec((1, k_nn, qb2, dp),
                         lambda i: (i // nq, 0, i % nq, 0)),
            pl.BlockSpec((1, k_nn, qb2, df),
                         lambda i: (i // nq, 0, i % nq, 0)),
            full((9, pos)), full((1, pos)),
            full((c0, e2)), full((1, e2)),
            full((e2, e)), full((1, e)),
            full((e, e)), full((1, e)),
            full((e, e)), full((1, e)),
            full((e, e)), full((1, e)),
            full((1, e)),
        ],
        out_specs=pl.BlockSpec((1, qb2, 1), lambda i: (i // nq, i % nq, 0)),
        out_shape=jax.ShapeDtypeStruct((b_dim, q_dim, 1), jnp.float32),
    )(query, gp4, gf4, wpos, bpos, w0, b0, w1, b1, w2, b2, w3, b3, w4,
      b4, wout_row)


def kernel(query, xyz, pcl_feat, W_pos, b_pos, W0, b0, W1, b1, W2, b2,
           W3, b3, W4, b4, Wout, bout):
    b_dim, q_dim, _ = query.shape
    n = xyz.shape[1]
    out_dim = pcl_feat.shape[-1]
    pos = W_pos.shape[1]
    k_nn = W0.shape[0] // (pos + out_dim)
    dp = 128

    xyzt = jnp.transpose(xyz, (0, 2, 1))
    xyzp = jnp.pad(xyz.reshape(b_dim * n, 3), ((0, 0), (0, dp - 3)))
    pf = pcl_feat.reshape(b_dim * n, out_dim)

    wargs = (
        W_pos.astype(jnp.bfloat16), b_pos.reshape(1, pos),
        W0.astype(jnp.bfloat16), b0.reshape(1, -1),
        W1.astype(jnp.bfloat16), b1.reshape(1, -1),
        W2.astype(jnp.bfloat16), b2.reshape(1, -1),
        W3.astype(jnp.bfloat16), b3.reshape(1, -1),
        W4.astype(jnp.bfloat16), b4.reshape(1, -1),
        Wout.reshape(1, -1).astype(jnp.bfloat16),
    )

    idxs = [
        _knn(query[b:b + 1], xyzt[b:b + 1], k_nn, qb=128, base=b * n)
        for b in range(b_dim)
    ]
    outs = []
    for b in range(b_dim):
        gp, gf = _sc_gather(idxs[b].reshape(-1), xyzp, pf)
        outs.append(_mlp(
            query[b:b + 1],
            gp.reshape(1, k_nn, q_dim, dp),
            gf.reshape(1, k_nn, q_dim, out_dim),
            *wargs, k_nn=k_nn, dp=dp, qb2=256,
        ))
    return jnp.concatenate(outs, axis=0) + bout


# speedup vs baseline: 40.8117x; 1.0251x over previous
import functools

import jax
import jax.numpy as jnp
from jax import lax
from jax.experimental import pallas as pl
from jax.experimental.pallas import tpu as pltpu
from jax.experimental.pallas import tpu_sc as plsc

_NC = 2
_NS = 16
_NW = _NC * _NS


def _bf(v):
    return v.astype(jnp.bfloat16).astype(jnp.float32)


def _knn_body(k_nn, n, base, q_ref, xt_ref, idx_ref):
    q = q_ref[0]
    qx, qy, qz = q[:, 0:1], q[:, 1:2], q[:, 2:3]
    xt = xt_ref[0]
    x0, x1, x2 = xt[0:1, :], xt[1:2, :], xt[2:3, :]

    qn = qx * qx + qy * qy + qz * qz
    xn = x0 * x0 + x1 * x1 + x2 * x2
    dot = jnp.dot(q.astype(jnp.bfloat16), xt.astype(jnp.bfloat16),
                  preferred_element_type=jnp.float32)
    d2 = (qn - 2.0 * dot) + xn

    qb = q.shape[0]
    inf = jnp.float32(jnp.inf)

    f_ch = 32
    s_sl = n // f_ch
    parts = [d2[:, f * s_sl:(f + 1) * s_sl] for f in range(f_ch)]

    def running_min(masked_parts):
        v = masked_parts[0]
        c = jnp.zeros((qb, s_sl), jnp.int32)
        for f in range(1, f_ch):
            lt = masked_parts[f] < v
            v = jnp.where(lt, masked_parts[f], v)
            c = jnp.where(lt, jnp.int32(f), c)
        return v, c

    v1, c1 = running_min(parts)
    parts2 = [jnp.where(c1 == f, inf, parts[f]) for f in range(f_ch)]
    v2, c2 = running_min(parts2)
    parts3 = [jnp.where(c2 == f, inf, parts2[f]) for f in range(f_ch)]
    v3, c3 = running_min(parts3)

    si = lax.broadcasted_iota(jnp.int32, (1, s_sl), 1)
    a_v, b_v, c_v = v1, v2, v3
    a_g = c1 * s_sl + si
    b_g = c2 * s_sl + si
    c_g = c3 * s_sl + si

    sent = jnp.int32(2 * n)
    depth = jnp.zeros((qb, s_sl), jnp.int32)
    bad = jnp.zeros((qb, 1), jnp.bool_)
    cols = []
    for j in range(k_nn):
        m = jnp.min(a_v, axis=1, keepdims=True)
        g = jnp.min(jnp.where(a_v == m, a_g, sent), axis=1, keepdims=True)
        cols.append(g)
        sel = (a_g == g) & (a_v == m)
        if j < k_nn - 1:
            bad = bad | jnp.any(sel & (depth == 2), axis=1, keepdims=True)
            depth = depth + sel.astype(jnp.int32)
            a_v = jnp.where(sel, b_v, a_v)
            a_g = jnp.where(sel, b_g, a_g)
            b_v = jnp.where(sel, c_v, b_v)
            b_g = jnp.where(sel, c_g, b_g)
            c_v = jnp.where(sel, inf, c_v)
    fast = jnp.concatenate(cols, axis=1)

    def slow_path(_):
        dd = d2
        iota = lax.broadcasted_iota(jnp.int32, (qb, n), 1)
        cc = []
        for _ in range(k_nn):
            ik = jnp.argmin(dd, axis=1).reshape(qb, 1)
            cc.append(ik)
            dd = jnp.where(iota == ik, inf, dd)
        return jnp.concatenate(cc, axis=1)

    idxs = lax.cond(jnp.any(bad), slow_path, lambda _: fast, None)
    idx_ref[0] = jnp.transpose(idxs + base, (1, 0))


def _knn(query, xyzt, k_nn, qb, base):
    b_dim, q_dim, _ = query.shape
    n = xyzt.shape[2]
    return pl.pallas_call(
        functools.partial(_knn_body, k_nn, n, base),
        grid=(b_dim, q_dim // qb),
        in_specs=[
            pl.BlockSpec((1, qb, 3), lambda b, q: (b, q, 0)),
            pl.BlockSpec((1, 3, n), lambda b, q: (b, 0, 0)),
        ],
        out_specs=pl.BlockSpec((1, k_nn, qb), lambda b, q: (b, 0, q)),
        out_shape=jax.ShapeDtypeStruct((b_dim, k_nn, q_dim), jnp.int32),
    )(query, xyzt)


def _sc_gather(idx_flat, tp, tf):
    total = idx_flat.shape[0]
    per_w = total // _NW
    chunk = 256
    nchunks = per_w // chunk
    dp = tp.shape[1]
    df = tf.shape[1]
    mesh = plsc.VectorSubcoreMesh(core_axis_name="c", subcore_axis_name="s")

    @functools.partial(
        pl.kernel,
        out_type=[
            jax.ShapeDtypeStruct((total, dp), jnp.float32),
            jax.ShapeDtypeStruct((total, df), jnp.float32),
        ],
        mesh=mesh,
        scratch_types=[
            pltpu.VMEM((per_w,), jnp.int32),
            pltpu.VMEM((chunk, dp), jnp.float32),
            pltpu.VMEM((chunk, df), jnp.float32),
            pltpu.SemaphoreType.DMA,
            pltpu.SemaphoreType.DMA,
        ],
    )
    def gather_k(idx_hbm, tp_hbm, tf_hbm, gp_hbm, gf_hbm,
                 idx_v, bufp, buff, semp, semf):
        wid = lax.axis_index("s") * _NC + lax.axis_index("c")
        base = wid * per_w
        pltpu.sync_copy(idx_hbm.at[pl.ds(base, per_w)], idx_v)
        for c in range(nchunks):
            sl = idx_v.at[pl.ds(c * chunk, chunk)]
            cp_p = pltpu.async_copy(tp_hbm.at[sl], bufp, semp)
            cp_f = pltpu.async_copy(tf_hbm.at[sl], buff, semf)
            cp_p.wait()
            cp_f.wait()
            pltpu.sync_copy(bufp, gp_hbm.at[pl.ds(base + c * chunk, chunk)])
            pltpu.sync_copy(buff, gf_hbm.at[pl.ds(base + c * chunk, chunk)])

    return gather_k(idx_flat, tp, tf)


def _mlp_body(k_nn, dp, pos, q_ref, gp_ref, gf_ref, wpos_ref, bpos_ref,
              w0_ref, b0_ref, w1_ref, b1_ref, w2_ref, b2_ref,
              w3_ref, b3_ref, w4_ref, b4_ref, wout_ref, out_ref):
    q = q_ref[0]
    qx, qy, qz = q[:, 0:1], q[:, 1:2], q[:, 2:3]
    wpos = wpos_ref[...].astype(jnp.float32)
    bpos = bpos_ref[...]

    pieces = []
    for k in range(k_nn):
        gpk = gp_ref[0, k]
        px, py, pz = gpk[:, 0:1], gpk[:, 1:2], gpk[:, 2:3]
        ins = (qx, qy, qz, px, py, pz, px - qx, py - qy, pz - qz)
        acc = bpos
        for d in range(9):
            acc = acc + _bf(ins[d]) * wpos[d:d + 1, :]
        posk = jnp.maximum(acc, 0.0).astype(jnp.bfloat16)
        pieces.append(posk)
        pieces.append(gf_ref[0, k].astype(jnp.bfloat16))
    x = jnp.concatenate(pieces, axis=1)

    def layer(h, w_ref, b_ref):
        hh = jnp.dot(h.astype(jnp.bfloat16), w_ref[...],
                     preferred_element_type=jnp.float32)
        return jnp.maximum(hh + b_ref[...], 0.0)

    h = layer(x, w0_ref, b0_ref)
    h = layer(h, w1_ref, b1_ref)
    h = layer(h, w2_ref, b2_ref)
    h = layer(h, w3_ref, b3_ref)
    h = layer(h, w4_ref, b4_ref)
    wo = wout_ref[...].astype(jnp.float32)
    out_ref[0] = jnp.sum(_bf(h) * wo, axis=1, keepdims=True)


def _mlp(query, gp4, gf4, wpos, bpos, w0, b0, w1, b1, w2, b2, w3, b3,
         w4, b4, wout_row, k_nn, dp, qb2):
    b_dim, q_dim, _ = query.shape
    pos = wpos.shape[1]
    df = gf4.shape[3]
    c0 = w0.shape[0]
    e2 = w0.shape[1]
    e = w1.shape[1]
    nq = q_dim // qb2
    full = lambda shape: pl.BlockSpec(shape, lambda i: tuple(0 for _ in shape))
    return pl.pallas_call(
        functools.partial(_mlp_body, k_nn, dp, pos),
        grid=(b_dim * nq,),
        in_specs=[
            pl.BlockSpec((1, qb2, 3), lambda i: (i // nq, i % nq, 0)),
            pl.BlockSpec((1, k_nn, qb2, dp),
                         lambda i: (i // nq, 0, i % nq, 0)),
            pl.BlockSpec((1, k_nn, qb2, df),
                         lambda i: (i // nq, 0, i % nq, 0)),
            full((9, pos)), full((1, pos)),
            full((c0, e2)), full((1, e2)),
            full((e2, e)), full((1, e)),
            full((e, e)), full((1, e)),
            full((e, e)), full((1, e)),
            full((e, e)), full((1, e)),
            full((1, e)),
        ],
        out_specs=pl.BlockSpec((1, qb2, 1), lambda i: (i // nq, i % nq, 0)),
        out_shape=jax.ShapeDtypeStruct((b_dim, q_dim, 1), jnp.float32),
    )(query, gp4, gf4, wpos, bpos, w0, b0, w1, b1, w2, b2, w3, b3, w4,
      b4, wout_row)


def kernel(query, xyz, pcl_feat, W_pos, b_pos, W0, b0, W1, b1, W2, b2,
           W3, b3, W4, b4, Wout, bout):
    b_dim, q_dim, _ = query.shape
    n = xyz.shape[1]
    out_dim = pcl_feat.shape[-1]
    pos = W_pos.shape[1]
    k_nn = W0.shape[0] // (pos + out_dim)
    dp = 128

    xyzt = jnp.transpose(xyz, (0, 2, 1))
    xyzp = jnp.pad(xyz.reshape(b_dim * n, 3), ((0, 0), (0, dp - 3)))
    pf = pcl_feat.reshape(b_dim * n, out_dim)

    wargs = (
        W_pos.astype(jnp.bfloat16), b_pos.reshape(1, pos),
        W0.astype(jnp.bfloat16), b0.reshape(1, -1),
        W1.astype(jnp.bfloat16), b1.reshape(1, -1),
        W2.astype(jnp.bfloat16), b2.reshape(1, -1),
        W3.astype(jnp.bfloat16), b3.reshape(1, -1),
        W4.astype(jnp.bfloat16), b4.reshape(1, -1),
        Wout.reshape(1, -1).astype(jnp.bfloat16),
    )

    idxs = [
        _knn(query[b:b + 1], xyzt[b:b + 1], k_nn, qb=128, base=b * n)
        for b in range(b_dim)
    ]
    outs = []
    for b in range(b_dim):
        gp, gf = _sc_gather(idxs[b].reshape(-1), xyzp, pf)
        outs.append(_mlp(
            query[b:b + 1],
            gp.reshape(1, k_nn, q_dim, dp),
            gf.reshape(1, k_nn, q_dim, out_dim),
            *wargs, k_nn=k_nn, dp=dp, qb2=256,
        ))
    return jnp.concatenate(outs, axis=0) + bout


# speedup vs baseline: 41.6888x; 1.0215x over previous
import functools

import jax
import jax.numpy as jnp
from jax import lax
from jax.experimental import pallas as pl
from jax.experimental.pallas import tpu as pltpu
from jax.experimental.pallas import tpu_sc as plsc

_NC = 2
_NS = 16
_NW = _NC * _NS


def _bf(v):
    return v.astype(jnp.bfloat16).astype(jnp.float32)


def _knn_body(k_nn, n, base, q_ref, xt_ref, idx_ref):
    q = q_ref[0]
    qx, qy, qz = q[:, 0:1], q[:, 1:2], q[:, 2:3]
    xt = xt_ref[0]
    x0, x1, x2 = xt[0:1, :], xt[1:2, :], xt[2:3, :]

    qn = qx * qx + qy * qy + qz * qz
    xn = x0 * x0 + x1 * x1 + x2 * x2
    dot = jnp.dot(q.astype(jnp.bfloat16), xt.astype(jnp.bfloat16),
                  preferred_element_type=jnp.float32)
    d2 = (qn - 2.0 * dot) + xn

    qb = q.shape[0]
    inf = jnp.float32(jnp.inf)

    f_ch = 32
    s_sl = n // f_ch
    parts = [d2[:, f * s_sl:(f + 1) * s_sl] for f in range(f_ch)]

    def running_min(masked_parts):
        v = masked_parts[0]
        c = jnp.zeros((qb, s_sl), jnp.int32)
        for f in range(1, f_ch):
            lt = masked_parts[f] < v
            v = jnp.where(lt, masked_parts[f], v)
            c = jnp.where(lt, jnp.int32(f), c)
        return v, c

    v1, c1 = running_min(parts)
    parts2 = [jnp.where(c1 == f, inf, parts[f]) for f in range(f_ch)]
    v2, c2 = running_min(parts2)
    parts3 = [jnp.where(c2 == f, inf, parts2[f]) for f in range(f_ch)]
    v3, c3 = running_min(parts3)

    si = lax.broadcasted_iota(jnp.int32, (1, s_sl), 1)
    a_v, b_v, c_v = v1, v2, v3
    a_g = c1 * s_sl + si
    b_g = c2 * s_sl + si
    c_g = c3 * s_sl + si

    sent = jnp.int32(2 * n)
    depth = jnp.zeros((qb, s_sl), jnp.int32)
    bad = jnp.zeros((qb, 1), jnp.bool_)
    cols = []
    for j in range(k_nn):
        m = jnp.min(a_v, axis=1, keepdims=True)
        g = jnp.min(jnp.where(a_v == m, a_g, sent), axis=1, keepdims=True)
        cols.append(g)
        sel = (a_g == g) & (a_v == m)
        if j < k_nn - 1:
            bad = bad | jnp.any(sel & (depth == 2), axis=1, keepdims=True)
            depth = depth + sel.astype(jnp.int32)
            a_v = jnp.where(sel, b_v, a_v)
            a_g = jnp.where(sel, b_g, a_g)
            b_v = jnp.where(sel, c_v, b_v)
            b_g = jnp.where(sel, c_g, b_g)
            c_v = jnp.where(sel, inf, c_v)
    fast = jnp.concatenate(cols, axis=1)

    def slow_path(_):
        dd = d2
        iota = lax.broadcasted_iota(jnp.int32, (qb, n), 1)
        cc = []
        for _ in range(k_nn):
            ik = jnp.argmin(dd, axis=1).reshape(qb, 1)
            cc.append(ik)
            dd = jnp.where(iota == ik, inf, dd)
        return jnp.concatenate(cc, axis=1)

    idxs = lax.cond(jnp.any(bad), slow_path, lambda _: fast, None)
    idx_ref[0] = jnp.transpose(idxs + base, (1, 0))


def _knn(query, xyzt, k_nn, qb, base):
    b_dim, q_dim, _ = query.shape
    n = xyzt.shape[2]
    return pl.pallas_call(
        functools.partial(_knn_body, k_nn, n, base),
        grid=(b_dim, q_dim // qb),
        in_specs=[
            pl.BlockSpec((1, qb, 3), lambda b, q: (b, q, 0)),
            pl.BlockSpec((1, 3, n), lambda b, q: (b, 0, 0)),
        ],
        out_specs=pl.BlockSpec((1, k_nn, qb), lambda b, q: (b, 0, q)),
        out_shape=jax.ShapeDtypeStruct((b_dim, k_nn, q_dim), jnp.int32),
    )(query, xyzt)


def _sc_gather(idx_flat, tp, tf):
    total = idx_flat.shape[0]
    per_w = total // _NW
    chunk = 256
    nchunks = per_w // chunk
    dp = tp.shape[1]
    df = tf.shape[1]
    mesh = plsc.VectorSubcoreMesh(core_axis_name="c", subcore_axis_name="s")

    @functools.partial(
        pl.kernel,
        out_type=[
            jax.ShapeDtypeStruct((total, dp), jnp.float32),
            jax.ShapeDtypeStruct((total, df), jnp.float32),
        ],
        mesh=mesh,
        scratch_types=[
            pltpu.VMEM((per_w,), jnp.int32),
            pltpu.VMEM((chunk, dp), jnp.float32),
            pltpu.VMEM((chunk, df), jnp.float32),
            pltpu.SemaphoreType.DMA,
            pltpu.SemaphoreType.DMA,
        ],
    )
    def gather_k(idx_hbm, tp_hbm, tf_hbm, gp_hbm, gf_hbm,
                 idx_v, bufp, buff, semp, semf):
        wid = lax.axis_index("s") * _NC + lax.axis_index("c")
        base = wid * per_w
        pltpu.sync_copy(idx_hbm.at[pl.ds(base, per_w)], idx_v)
        for c in range(nchunks):
            sl = idx_v.at[pl.ds(c * chunk, chunk)]
            cp_p = pltpu.async_copy(tp_hbm.at[sl], bufp, semp)
            cp_f = pltpu.async_copy(tf_hbm.at[sl], buff, semf)
            cp_p.wait()
            cp_f.wait()
            pltpu.sync_copy(bufp, gp_hbm.at[pl.ds(base + c * chunk, chunk)])
            pltpu.sync_copy(buff, gf_hbm.at[pl.ds(base + c * chunk, chunk)])

    return gather_k(idx_flat, tp, tf)


def _mlp_body(k_nn, dp, pos, q_ref, gp_ref, gf_ref, wpos_ref, bpos_ref,
              w0_ref, b0_ref, w1_ref, b1_ref, w2_ref, b2_ref,
              w3_ref, b3_ref, w4_ref, b4_ref, wout_ref, out_ref):
    q = q_ref[0]
    qx, qy, qz = q[:, 0:1], q[:, 1:2], q[:, 2:3]
    wpos = wpos_ref[...]
    bpos = bpos_ref[...]

    pieces = []
    for k in range(k_nn):
        gpk = gp_ref[0, k]
        px, py, pz = gpk[:, 0:1], gpk[:, 1:2], gpk[:, 2:3]
        ins = jnp.concatenate(
            (qx, qy, qz, px, py, pz, px - qx, py - qy, pz - qz), axis=1)
        acc = jnp.dot(ins.astype(jnp.bfloat16), wpos,
                      preferred_element_type=jnp.float32) + bpos
        posk = jnp.maximum(acc, 0.0).astype(jnp.bfloat16)
        pieces.append(posk)
        pieces.append(gf_ref[0, k].astype(jnp.bfloat16))
    x = jnp.concatenate(pieces, axis=1)

    def layer(h, w_ref, b_ref):
        hh = jnp.dot(h.astype(jnp.bfloat16), w_ref[...],
                     preferred_element_type=jnp.float32)
        return jnp.maximum(hh + b_ref[...], 0.0)

    h = layer(x, w0_ref, b0_ref)
    h = layer(h, w1_ref, b1_ref)
    h = layer(h, w2_ref, b2_ref)
    h = layer(h, w3_ref, b3_ref)
    h = layer(h, w4_ref, b4_ref)
    wo = wout_ref[...].astype(jnp.float32)
    out_ref[0] = jnp.sum(_bf(h) * wo, axis=1, keepdims=True)


def _mlp(query, gp4, gf4, wpos, bpos, w0, b0, w1, b1, w2, b2, w3, b3,
         w4, b4, wout_row, k_nn, dp, qb2):
    b_dim, q_dim, _ = query.shape
    pos = wpos.shape[1]
    df = gf4.shape[3]
    c0 = w0.shape[0]
    e2 = w0.shape[1]
    e = w1.shape[1]
    nq = q_dim // qb2
    full = lambda shape: pl.BlockSpec(shape, lambda i: tuple(0 for _ in shape))
    return pl.pallas_call(
        functools.partial(_mlp_body, k_nn, dp, pos),
        grid=(b_dim * nq,),
        in_specs=[
            pl.BlockSpec((1, qb2, 3), lambda i: (i // nq, i % nq, 0)),
            pl.BlockSpec((1, k_nn, qb2, dp),
                         lambda i: (i // nq, 0, i % nq, 0)),
            pl.BlockSpec((1, k_nn, qb2, df),
                         lambda i: (i // nq, 0, i % nq, 0)),
            full((9, pos)), full((1, pos)),
            full((c0, e2)), full((1, e2)),
            full((e2, e)), full((1, e)),
            full((e, e)), full((1, e)),
            full((e, e)), full((1, e)),
            full((e, e)), full((1, e)),
            full((1, e)),
        ],
        out_specs=pl.BlockSpec((1, qb2, 1), lambda i: (i // nq, i % nq, 0)),
        out_shape=jax.ShapeDtypeStruct((b_dim, q_dim, 1), jnp.float32),
    )(query, gp4, gf4, wpos, bpos, w0, b0, w1, b1, w2, b2, w3, b3, w4,
      b4, wout_row)


def kernel(query, xyz, pcl_feat, W_pos, b_pos, W0, b0, W1, b1, W2, b2,
           W3, b3, W4, b4, Wout, bout):
    b_dim, q_dim, _ = query.shape
    n = xyz.shape[1]
    out_dim = pcl_feat.shape[-1]
    pos = W_pos.shape[1]
    k_nn = W0.shape[0] // (pos + out_dim)
    dp = 128

    xyzt = jnp.transpose(xyz, (0, 2, 1))
    xyzp = jnp.pad(xyz.reshape(b_dim * n, 3), ((0, 0), (0, dp - 3)))
    pf = pcl_feat.reshape(b_dim * n, out_dim)

    wargs = (
        W_pos.astype(jnp.bfloat16), b_pos.reshape(1, pos),
        W0.astype(jnp.bfloat16), b0.reshape(1, -1),
        W1.astype(jnp.bfloat16), b1.reshape(1, -1),
        W2.astype(jnp.bfloat16), b2.reshape(1, -1),
        W3.astype(jnp.bfloat16), b3.reshape(1, -1),
        W4.astype(jnp.bfloat16), b4.reshape(1, -1),
        Wout.reshape(1, -1).astype(jnp.bfloat16),
    )

    idxs = [
        _knn(query[b:b + 1], xyzt[b:b + 1], k_nn, qb=128, base=b * n)
        for b in range(b_dim)
    ]
    outs = []
    for b in range(b_dim):
        gp, gf = _sc_gather(idxs[b].reshape(-1), xyzp, pf)
        outs.append(_mlp(
            query[b:b + 1],
            gp.reshape(1, k_nn, q_dim, dp),
            gf.reshape(1, k_nn, q_dim, out_dim),
            *wargs, k_nn=k_nn, dp=dp, qb2=256,
        ))
    return jnp.concatenate(outs, axis=0) + bout
